# Initial kernel scaffold; baseline (speedup 1.0000x reference)
import jax
import jax.numpy as jnp
from jax.experimental import pallas as pl


def kernel(rel_vec, hash_weight):
    raise NotImplementedError("write your pallas kernel here")


# speedup vs baseline: 1.3239x; 1.3239x over previous
import functools

import numpy as np
import jax
import jax.numpy as jnp
from jax import lax
from jax.experimental import pallas as pl
from jax.experimental.pallas import tpu as pltpu
from jax.experimental.pallas import tpu_sc as plsc

_MEM_SIZE = (2, 10, 10, 2, 10, 10, 2, 10, 10, 2)
_N_CELLS = 80000
_SLAB = 200
_N = 207872
_RV_W = 241

_ROWS = 1024
_N_BLOCKS = _N // _ROWS

_CONSTS = np.zeros((2, 128), np.int32)
_CONSTS[0, :7] = [40000, 4000, 400, 200, 20, 2, 1]
_CONSTS[1, :7] = [1, 9, 9, 1, 9, 9, 1]


def _hash_body(x_ref, w_ref, c_ref, o_ref):
    h = lax.dot_general(
        x_ref[...], w_ref[...], (((1,), (0,)), ((), ())),
        preferred_element_type=jnp.float32,
    )
    hi = h.astype(jnp.int32)
    hc = jnp.minimum(jnp.maximum(hi, 0), c_ref[1, :][None, :])
    lin = jnp.sum(hc * c_ref[0, :][None, :], axis=1, dtype=jnp.int32)
    o_ref[...] = lin.reshape(1, _ROWS // 128, 128)


_hash_call = pl.pallas_call(
    _hash_body,
    grid=(_N_BLOCKS,),
    in_specs=[
        pl.BlockSpec((_ROWS, _RV_W), lambda i: (i, 0)),
        pl.BlockSpec((_RV_W, 128), lambda i: (0, 0)),
        pl.BlockSpec((2, 128), lambda i: (0, 0)),
    ],
    out_specs=pl.BlockSpec((1, _ROWS // 128, 128), lambda i: (i, 0, 0)),
    out_shape=jax.ShapeDtypeStruct((_N_BLOCKS, _ROWS // 128, 128), jnp.int32),
)

_NC, _NS = 2, 16
_NW = _NC * _NS
_PER_W = _N // _NW
_CHUNK = 112
_NCHUNK = _PER_W // _CHUNK
_TAB = _NS * 5120
_ZSL = 5120
_OSL = _N_CELLS // _NS

def _hist_body(lin_hbm, out_hbm, idx_v, ones_v, stage_v, table_sp):
    cid = lax.axis_index("c")
    sid = lax.axis_index("s")
    wid = cid * _NS + sid

    for k in range(_CHUNK // 16):
        ones_v[pl.ds(k * 16, 16)] = jnp.full((16,), 1.0, jnp.float32)

    def _zero(i, _):
        stage_v[pl.ds(i * 16, 16)] = jnp.zeros((16,), jnp.float32)
        return 0

    lax.fori_loop(0, _ZSL // 16, _zero, 0)
    pltpu.sync_copy(stage_v, table_sp.at[pl.ds(sid * _ZSL, _ZSL)])
    pltpu.sync_copy(lin_hbm.at[wid], idx_v)
    plsc.subcore_barrier()

    def _scat(j, _):
        pltpu.sync_copy(ones_v, table_sp.at[idx_v.at[j]], add=True)
        return 0

    lax.fori_loop(0, _NCHUNK, _scat, 0)
    plsc.subcore_barrier()

    pltpu.sync_copy(table_sp.at[pl.ds(sid * _OSL, _OSL)],
                    stage_v.at[pl.ds(0, _OSL)])
    pltpu.sync_copy(stage_v.at[pl.ds(0, _OSL)],
                    out_hbm.at[pl.ds(cid * _N_CELLS + sid * _OSL, _OSL)])


@functools.cache
def _make_hist_call():
    mesh = plsc.VectorSubcoreMesh(
        core_axis_name="c", subcore_axis_name="s",
        num_cores=_NC, num_subcores=_NS,
    )
    return pl.kernel(
        _hist_body,
        out_type=jax.ShapeDtypeStruct((_NC * _N_CELLS,), jnp.float32),
        mesh=mesh,
        scratch_types=[
            pltpu.VMEM((_NCHUNK, _CHUNK), jnp.int32),
            pltpu.VMEM((_CHUNK,), jnp.float32),
            pltpu.VMEM((_ZSL,), jnp.float32),
            pltpu.VMEM_SHARED((_TAB,), jnp.float32),
        ],
    )


_RB = 3200


def _bcast_body(c_ref, o_ref):
    c = c_ref[0, :] + c_ref[1, :]
    o_ref[...] = jnp.broadcast_to(c[:, None], (_RB, _SLAB))


_bcast_call = pl.pallas_call(
    _bcast_body,
    grid=(_N_CELLS // _RB,),
    in_specs=[pl.BlockSpec((_NC, _RB), lambda i: (0, i))],
    out_specs=pl.BlockSpec((_RB, _SLAB), lambda i: (i, 0)),
    out_shape=jax.ShapeDtypeStruct((_N_CELLS, _SLAB), jnp.float32),
)


def kernel(rel_vec, hash_weight):
    w_pad = jnp.zeros((_RV_W, 128), jnp.float32).at[:, :7].set(hash_weight.T)
    lin = _hash_call(rel_vec, w_pad, jnp.asarray(_CONSTS))
    lin = lin.reshape(_NW, _NCHUNK, _CHUNK)
    counts = _make_hist_call()(lin).reshape(_NC, _N_CELLS)
    out = _bcast_call(counts)
    return out.reshape(_MEM_SIZE)


# speedup vs baseline: 47.8589x; 36.1493x over previous
import functools

import numpy as np
import jax
import jax.numpy as jnp
from jax import lax
from jax.experimental import pallas as pl
from jax.experimental.pallas import tpu as pltpu
from jax.experimental.pallas import tpu_sc as plsc

_MEM_SIZE = (2, 10, 10, 2, 10, 10, 2, 10, 10, 2)
_N_CELLS = 80000
_SLAB = 200
_N = 207872
_RV_W = 241

_ROWS = 1024
_N_BLOCKS = _N // _ROWS

_CONSTS = np.zeros((2, 128), np.int32)
_CONSTS[0, :7] = [40000, 4000, 400, 200, 20, 2, 1]
_CONSTS[1, :7] = [1, 9, 9, 1, 9, 9, 1]


def _hash_body(x_ref, w_ref, c_ref, o_ref):
    h = lax.dot_general(
        x_ref[...], w_ref[...], (((0,), (0,)), ((), ())),
        preferred_element_type=jnp.float32,
    )
    hi = h.astype(jnp.int32)
    hc = jnp.minimum(jnp.maximum(hi, 0), c_ref[1, :][None, :])
    lin = jnp.sum(hc * c_ref[0, :][None, :], axis=1, dtype=jnp.int32)
    o_ref[...] = lin.reshape(1, _ROWS // 128, 128)


_hash_call = pl.pallas_call(
    _hash_body,
    grid=(_N_BLOCKS,),
    in_specs=[
        pl.BlockSpec((_RV_W, _ROWS), lambda i: (0, i)),
        pl.BlockSpec((_RV_W, 128), lambda i: (0, 0)),
        pl.BlockSpec((2, 128), lambda i: (0, 0)),
    ],
    out_specs=pl.BlockSpec((1, _ROWS // 128, 128), lambda i: (i, 0, 0)),
    out_shape=jax.ShapeDtypeStruct((_N_BLOCKS, _ROWS // 128, 128), jnp.int32),
)

_NC, _NS = 2, 16
_NW = _NC * _NS
_PER_W = _N // _NW
_CHUNK = 112
_NCHUNK = _PER_W // _CHUNK
_TAB = _NS * 5120
_ZSL = 5120
_OSL = _N_CELLS // _NS

def _hist_body(lin_hbm, out_hbm, idx_v, ones_v, stage_v, table_sp):
    cid = lax.axis_index("c")
    sid = lax.axis_index("s")
    wid = cid * _NS + sid

    for k in range(_CHUNK // 16):
        ones_v[pl.ds(k * 16, 16)] = jnp.full((16,), 1.0, jnp.float32)

    def _zero(i, _):
        stage_v[pl.ds(i * 16, 16)] = jnp.zeros((16,), jnp.float32)
        return 0

    lax.fori_loop(0, _ZSL // 16, _zero, 0)
    pltpu.sync_copy(stage_v, table_sp.at[pl.ds(sid * _ZSL, _ZSL)])
    pltpu.sync_copy(lin_hbm.at[wid], idx_v)
    plsc.subcore_barrier()

    def _scat(j, _):
        pltpu.sync_copy(ones_v, table_sp.at[idx_v.at[j]], add=True)
        return 0

    lax.fori_loop(0, _NCHUNK, _scat, 0)
    plsc.subcore_barrier()

    pltpu.sync_copy(table_sp.at[pl.ds(sid * _OSL, _OSL)],
                    stage_v.at[pl.ds(0, _OSL)])
    pltpu.sync_copy(stage_v.at[pl.ds(0, _OSL)],
                    out_hbm.at[pl.ds(cid * _N_CELLS + sid * _OSL, _OSL)])


@functools.cache
def _make_hist_call():
    mesh = plsc.VectorSubcoreMesh(
        core_axis_name="c", subcore_axis_name="s",
        num_cores=_NC, num_subcores=_NS,
    )
    return pl.kernel(
        _hist_body,
        out_type=jax.ShapeDtypeStruct((_NC * _N_CELLS,), jnp.float32),
        mesh=mesh,
        scratch_types=[
            pltpu.VMEM((_NCHUNK, _CHUNK), jnp.int32),
            pltpu.VMEM((_CHUNK,), jnp.float32),
            pltpu.VMEM((_ZSL,), jnp.float32),
            pltpu.VMEM_SHARED((_TAB,), jnp.float32),
        ],
    )


def kernel(rel_vec, hash_weight):
    w_pad = jnp.zeros((_RV_W, 128), jnp.float32).at[:, :7].set(hash_weight.T)
    lin = _hash_call(rel_vec.T, w_pad, jnp.asarray(_CONSTS))
    lin = lin.reshape(_NW, _NCHUNK, _CHUNK)
    counts = _make_hist_call()(lin).reshape(_NC, _N_CELLS)
    csum = (counts[0] + counts[1]).reshape(_MEM_SIZE[:7])
    return jnp.broadcast_to(csum[..., None, None, None], _MEM_SIZE)


# speedup vs baseline: 52.0145x; 1.0868x over previous
import functools

import numpy as np
import jax
import jax.numpy as jnp
from jax import lax
from jax.experimental import pallas as pl
from jax.experimental.pallas import tpu as pltpu
from jax.experimental.pallas import tpu_sc as plsc

_MEM_SIZE = (2, 10, 10, 2, 10, 10, 2, 10, 10, 2)
_N_CELLS = 80000
_SLAB = 200
_N = 207872
_RV_W = 241

_ROWS = 1792
_N_BLOCKS = _N // _ROWS

_CONSTS = np.zeros((2, 128), np.int32)
_CONSTS[0, :7] = [40000, 4000, 400, 200, 20, 2, 1]
_CONSTS[1, :7] = [1, 9, 9, 1, 9, 9, 1]


def _hash_body(x_ref, w_ref, c_ref, o_ref):
    h = lax.dot_general(
        x_ref[...], w_ref[...], (((0,), (0,)), ((), ())),
        preferred_element_type=jnp.float32,
    )
    hi = h.astype(jnp.int32)
    hc = jnp.minimum(jnp.maximum(hi, 0), c_ref[1, :][None, :])
    lin = jnp.sum(hc * c_ref[0, :][None, :], axis=1, dtype=jnp.int32)
    o_ref[...] = lin.reshape(1, _ROWS // 128, 128)


_hash_call = pl.pallas_call(
    _hash_body,
    grid=(_N_BLOCKS,),
    in_specs=[
        pl.BlockSpec((_RV_W, _ROWS), lambda i: (0, i)),
        pl.BlockSpec((_RV_W, 128), lambda i: (0, 0)),
        pl.BlockSpec((2, 128), lambda i: (0, 0)),
    ],
    out_specs=pl.BlockSpec((1, _ROWS // 128, 128), lambda i: (i, 0, 0)),
    out_shape=jax.ShapeDtypeStruct((_N_BLOCKS, _ROWS // 128, 128), jnp.int32),
)

_NC, _NS = 2, 16
_NW = _NC * _NS
_PER_W = _N // _NW
_CHUNK = 112
_NCHUNK = _PER_W // _CHUNK
_TAB = _NS * 5120
_ZSL = 5120
_OSL = _N_CELLS // _NS

def _hist_body(lin_hbm, out_hbm, idx_v, ones_v, stage_v, table_sp):
    cid = lax.axis_index("c")
    sid = lax.axis_index("s")
    wid = cid * _NS + sid

    for k in range(_CHUNK // 16):
        ones_v[pl.ds(k * 16, 16)] = jnp.full((16,), 1.0, jnp.float32)

    def _zero(i, _):
        stage_v[pl.ds(i * 16, 16)] = jnp.zeros((16,), jnp.float32)
        return 0

    lax.fori_loop(0, _ZSL // 16, _zero, 0)
    pltpu.sync_copy(stage_v, table_sp.at[pl.ds(sid * _ZSL, _ZSL)])
    pltpu.sync_copy(lin_hbm.at[wid], idx_v)
    plsc.subcore_barrier()

    def _scat(j, _):
        pltpu.sync_copy(ones_v, table_sp.at[idx_v.at[j]], add=True)
        return 0

    lax.fori_loop(0, _NCHUNK, _scat, 0)
    plsc.subcore_barrier()

    pltpu.sync_copy(table_sp.at[pl.ds(sid * _OSL, _OSL)],
                    stage_v.at[pl.ds(0, _OSL)])
    pltpu.sync_copy(stage_v.at[pl.ds(0, _OSL)],
                    out_hbm.at[pl.ds(cid * _N_CELLS + sid * _OSL, _OSL)])


@functools.cache
def _make_hist_call():
    mesh = plsc.VectorSubcoreMesh(
        core_axis_name="c", subcore_axis_name="s",
        num_cores=_NC, num_subcores=_NS,
    )
    return pl.kernel(
        _hist_body,
        out_type=jax.ShapeDtypeStruct((_NC * _N_CELLS,), jnp.float32),
        mesh=mesh,
        scratch_types=[
            pltpu.VMEM((_NCHUNK, _CHUNK), jnp.int32),
            pltpu.VMEM((_CHUNK,), jnp.float32),
            pltpu.VMEM((_ZSL,), jnp.float32),
            pltpu.VMEM_SHARED((_TAB,), jnp.float32),
        ],
    )


def kernel(rel_vec, hash_weight):
    w_pad = jnp.zeros((_RV_W, 128), jnp.float32).at[:, :7].set(hash_weight.T)
    lin = _hash_call(rel_vec.T, w_pad, jnp.asarray(_CONSTS))
    lin = lin.reshape(_NW, _NCHUNK, _CHUNK)
    counts = _make_hist_call()(lin).reshape(_NC, _N_CELLS)
    csum = (counts[0] + counts[1]).reshape(_MEM_SIZE[:7])
    return jnp.broadcast_to(csum[..., None, None, None], _MEM_SIZE)


# speedup vs baseline: 55.9756x; 1.0762x over previous
import functools

import numpy as np
import jax
import jax.numpy as jnp
from jax import lax
from jax.experimental import pallas as pl
from jax.experimental.pallas import tpu as pltpu
from jax.experimental.pallas import tpu_sc as plsc

_MEM_SIZE = (2, 10, 10, 2, 10, 10, 2, 10, 10, 2)
_N_CELLS = 80000
_SLAB = 200
_N = 207872
_RV_W = 241

_ROWS = 7168
_N_BLOCKS = _N // _ROWS

_CONSTS = np.zeros((2, 128), np.int32)
_CONSTS[0, :7] = [40000, 4000, 400, 200, 20, 2, 1]
_CONSTS[1, :7] = [1, 9, 9, 1, 9, 9, 1]


def _hash_body(x_ref, w_ref, c_ref, o_ref):
    h = lax.dot_general(
        x_ref[...], w_ref[...], (((0,), (0,)), ((), ())),
        preferred_element_type=jnp.float32,
    )
    hi = h.astype(jnp.int32)
    hc = jnp.minimum(jnp.maximum(hi, 0), c_ref[1, :][None, :])
    lin = jnp.sum(hc * c_ref[0, :][None, :], axis=1, dtype=jnp.int32)
    o_ref[...] = lin.reshape(1, _ROWS // 128, 128)


_hash_call = pl.pallas_call(
    _hash_body,
    grid=(_N_BLOCKS,),
    in_specs=[
        pl.BlockSpec((_RV_W, _ROWS), lambda i: (0, i)),
        pl.BlockSpec((_RV_W, 128), lambda i: (0, 0)),
        pl.BlockSpec((2, 128), lambda i: (0, 0)),
    ],
    out_specs=pl.BlockSpec((1, _ROWS // 128, 128), lambda i: (i, 0, 0)),
    out_shape=jax.ShapeDtypeStruct((_N_BLOCKS, _ROWS // 128, 128), jnp.int32),
)

_NC, _NS = 2, 16
_NW = _NC * _NS
_PER_W = _N // _NW
_CHUNK = 112
_NCHUNK = _PER_W // _CHUNK
_TAB = _NS * 5120
_ZSL = 5120
_OSL = _N_CELLS // _NS

def _hist_body(lin_hbm, out_hbm, idx_v, ones_v, stage_v, table_sp):
    cid = lax.axis_index("c")
    sid = lax.axis_index("s")
    wid = cid * _NS + sid

    for k in range(_CHUNK // 16):
        ones_v[pl.ds(k * 16, 16)] = jnp.full((16,), 1.0, jnp.float32)

    def _zero(i, _):
        stage_v[pl.ds(i * 16, 16)] = jnp.zeros((16,), jnp.float32)
        return 0

    lax.fori_loop(0, _ZSL // 16, _zero, 0)
    pltpu.sync_copy(stage_v, table_sp.at[pl.ds(sid * _ZSL, _ZSL)])
    pltpu.sync_copy(lin_hbm.at[wid], idx_v)
    plsc.subcore_barrier()

    def _scat(j, _):
        pltpu.sync_copy(ones_v, table_sp.at[idx_v.at[j]], add=True)
        return 0

    lax.fori_loop(0, _NCHUNK, _scat, 0)
    plsc.subcore_barrier()

    pltpu.sync_copy(table_sp.at[pl.ds(sid * _OSL, _OSL)],
                    stage_v.at[pl.ds(0, _OSL)])
    pltpu.sync_copy(stage_v.at[pl.ds(0, _OSL)],
                    out_hbm.at[pl.ds(cid * _N_CELLS + sid * _OSL, _OSL)])


@functools.cache
def _make_hist_call():
    mesh = plsc.VectorSubcoreMesh(
        core_axis_name="c", subcore_axis_name="s",
        num_cores=_NC, num_subcores=_NS,
    )
    return pl.kernel(
        _hist_body,
        out_type=jax.ShapeDtypeStruct((_NC * _N_CELLS,), jnp.float32),
        mesh=mesh,
        scratch_types=[
            pltpu.VMEM((_NCHUNK, _CHUNK), jnp.int32),
            pltpu.VMEM((_CHUNK,), jnp.float32),
            pltpu.VMEM((_ZSL,), jnp.float32),
            pltpu.VMEM_SHARED((_TAB,), jnp.float32),
        ],
    )


def kernel(rel_vec, hash_weight):
    w_pad = jnp.zeros((_RV_W, 128), jnp.float32).at[:, :7].set(hash_weight.T)
    lin = _hash_call(rel_vec.T, w_pad, jnp.asarray(_CONSTS))
    lin = lin.reshape(_NW, _NCHUNK, _CHUNK)
    counts = _make_hist_call()(lin).reshape(_NC, _N_CELLS)
    csum = (counts[0] + counts[1]).reshape(_MEM_SIZE[:7])
    return jnp.broadcast_to(csum[..., None, None, None], _MEM_SIZE)
